# baseline (speedup 1.0000x reference)
import jax
import jax.numpy as jnp
from jax.experimental import pallas as pl

B, H, W, C = 2, 224, 224, 96
G, KK = 4, 3
K2 = KK * KK
GC = C // G
GK = G * K2
OFF_SCALE = 0.1
BH = 16
N = H // BH


def _dcn_kernel(xc_ref, xh_ref, Win_ref, bin_ref, dw_ref, dwb_ref,
                pw_ref, pwb_ref, Wom_ref, bom_ref, Wout_ref, bout_ref, o_ref):
    i = pl.program_id(1)
    f32 = jnp.float32

    raw = jnp.concatenate([xh_ref[0, 0, 0:2], xc_ref[0], xh_ref[0, 0, 2:4]],
                          axis=0)
    xs2 = raw.reshape((BH + 4) * W, C)
    x_proj = (jnp.dot(xs2, Win_ref[...], preferred_element_type=f32)
              + bin_ref[...]).reshape(BH + 4, W, C)
    xpp = jnp.concatenate([x_proj[:, :1], x_proj[:, :1], x_proj,
                           x_proj[:, W - 1:], x_proj[:, W - 1:]], axis=1)

    xc = raw[1:BH + 3]
    g_io = jax.lax.broadcasted_iota(jnp.int32, (BH + 2, 1, 1), 0) + i * BH - 1
    xc = xc * jnp.logical_and(g_io >= 0, g_io < H).astype(f32)
    zcol = jnp.zeros((BH + 2, 1, C), f32)
    xcp = jnp.concatenate([zcol, xc, zcol], axis=1)
    dw = dw_ref[...]
    h = xcp[0:BH, 0:W] * dw[0]
    for kpos in range(1, 9):
        di, dj = kpos // 3, kpos % 3
        h = h + xcp[di:di + BH, dj:dj + W] * dw[kpos]
    h = h + dwb_ref[...]
    h = h * jax.nn.sigmoid(h)
    x_dw = jnp.dot(h.reshape(BH * W, C), pw_ref[...],
                   preferred_element_type=f32) + pwb_ref[...]

    om = (jnp.dot(x_dw, Wom_ref[...], preferred_element_type=f32)
          + bom_ref[...]).reshape(BH, W, 2 * GK + GK)
    logits = om[..., 2 * GK:3 * GK]

    e_raw = jnp.exp(logits - jnp.max(logits, axis=-1, keepdims=True))
    j36 = jax.lax.broadcasted_iota(jnp.int32, (GK, GK), 0) // K2
    c36 = jax.lax.broadcasted_iota(jnp.int32, (GK, GK), 1) // K2
    sel36 = (j36 == c36).astype(f32)
    norm36 = jnp.dot(e_raw.reshape(BH * W, GK), sel36,
                     preferred_element_type=f32).reshape(BH, W, GK)
    e_attn = e_raw / norm36

    k_io = jax.lax.broadcasted_iota(jnp.int32, (1, 1, 2 * GK), 2)
    d_sel = k_io >= GK
    k9 = k_io % K2
    gcoord = jnp.where(d_sel, k9 % 3 - 1, k9 // 3 - 1).astype(f32)
    p = om[..., 0:2 * GK] * OFF_SCALE + gcoord

    whw = [jnp.maximum(1.0 - jnp.abs(p - d), 0.0)
           for d in (-2.0, -1.0, 0.0, 1.0, 2.0)]
    wh = [a[..., 0:GK] for a in whw]
    ww = [a[..., GK:2 * GK] for a in whw]

    j_io = jax.lax.broadcasted_iota(jnp.int32, (GK, C), 0) // K2
    c_io = jax.lax.broadcasted_iota(jnp.int32, (GK, C), 1) // GC
    sel = (j_io == c_io).astype(f32)
    dy_parts = []
    for dyi in range(5):
        awh = e_attn * wh[dyi]
        terms = []
        for dxi in range(5):
            t = (awh * ww[dxi]).reshape(BH * W, GK)
            wt96 = jnp.dot(t, sel, preferred_element_type=f32).reshape(BH, W, C)
            terms.append(wt96 * xpp[dyi:dyi + BH, dxi:dxi + W])
        dy_parts.append(((terms[0] + terms[1]) + (terms[2] + terms[3]))
                        + terms[4])
    out_pre = ((dy_parts[0] + dy_parts[1]) + (dy_parts[2] + dy_parts[3])) + dy_parts[4]
    out = (jnp.dot(out_pre.reshape(BH * W, C), Wout_ref[...],
                   preferred_element_type=f32) + bout_ref[...]).reshape(BH, W, C)
    o_ref[0] = out


def kernel(x, W_in, b_in, W_out, b_out, dw_w, dw_b, pw_w, pw_b,
           W_off, b_off, W_mask, b_mask):
    f32 = jnp.float32
    Wofp = W_off.reshape(C, G, K2, 2).transpose(0, 3, 1, 2).reshape(C, 2 * GK)
    bofp = b_off.reshape(G, K2, 2).transpose(2, 0, 1).reshape(2 * GK)
    Wom = jnp.concatenate([Wofp, W_mask], axis=1)
    bom = jnp.concatenate([bofp, b_mask]).reshape(1, 3 * GK)

    hidx = []
    for i in range(N):
        hidx += [max(i * BH - 2, 0), max(i * BH - 1, 0),
                 min(i * BH + BH, H - 1), min(i * BH + BH + 1, H - 1)]
    xh = x[:, jnp.asarray(hidx)].reshape(B, N, 4, W, C)

    args = (
        x, xh,
        W_in.astype(f32), b_in.reshape(1, C).astype(f32),
        dw_w.reshape(K2, C).astype(f32), dw_b.reshape(1, C).astype(f32),
        pw_w.reshape(C, C).astype(f32), pw_b.reshape(1, C).astype(f32),
        Wom.astype(f32), bom.astype(f32),
        W_out.astype(f32), b_out.reshape(1, C).astype(f32),
    )

    def full(a):
        r = a.ndim
        return pl.BlockSpec(a.shape, lambda b, i, _r=r: (0,) * _r)

    in_specs = [
        pl.BlockSpec((1, BH, W, C), lambda b, i: (b, i, 0, 0)),
        pl.BlockSpec((1, 1, 4, W, C), lambda b, i: (b, i, 0, 0, 0)),
    ] + [full(a) for a in args[2:]]

    return pl.pallas_call(
        _dcn_kernel,
        grid=(B, N),
        in_specs=in_specs,
        out_specs=pl.BlockSpec((1, BH, W, C), lambda b, i: (b, i, 0, 0)),
        out_shape=jax.ShapeDtypeStruct((B, H, W, C), f32),
    )(*args)


# speedup vs baseline: 1.0308x; 1.0308x over previous
import jax
import jax.numpy as jnp
from jax.experimental import pallas as pl

B, H, W, C = 2, 224, 224, 96
G, KK = 4, 3
K2 = KK * KK
GC = C // G
GK = G * K2
OFF_SCALE = 0.1
BH = 16
N = H // BH


def _dcn_kernel(xp_ref, xc_ref, xn_ref, Win_ref, bin_ref, dw_ref, dwb_ref,
                pw_ref, pwb_ref, Wom_ref, bom_ref, Wout_ref, bout_ref, o_ref):
    i = pl.program_id(1)
    f32 = jnp.float32

    raw = jnp.concatenate([xp_ref[0, BH - 2:BH], xc_ref[0], xn_ref[0, 0:2]],
                          axis=0)

    r_io = jax.lax.broadcasted_iota(jnp.int32, (BH + 4, 1, 1), 0)
    xs = jnp.where(jnp.logical_and(i == 0, r_io < 2), raw[2:3], raw)
    xs = jnp.where(jnp.logical_and(i == N - 1, r_io > BH + 1), raw[BH + 1:BH + 2], xs)
    xs2 = xs.reshape((BH + 4) * W, C)
    x_proj = (jnp.dot(xs2, Win_ref[...], preferred_element_type=f32)
              + bin_ref[...]).reshape(BH + 4, W, C)
    xpp = jnp.concatenate([x_proj[:, :1], x_proj[:, :1], x_proj,
                           x_proj[:, W - 1:], x_proj[:, W - 1:]], axis=1)

    xc = raw[1:BH + 3]
    g_io = jax.lax.broadcasted_iota(jnp.int32, (BH + 2, 1, 1), 0) + i * BH - 1
    xc = xc * jnp.logical_and(g_io >= 0, g_io < H).astype(f32)
    zcol = jnp.zeros((BH + 2, 1, C), f32)
    xcp = jnp.concatenate([zcol, xc, zcol], axis=1)
    dw = dw_ref[...]
    h = xcp[0:BH, 0:W] * dw[0]
    for kpos in range(1, 9):
        di, dj = kpos // 3, kpos % 3
        h = h + xcp[di:di + BH, dj:dj + W] * dw[kpos]
    h = h + dwb_ref[...]
    h = h * jax.nn.sigmoid(h)
    x_dw = jnp.dot(h.reshape(BH * W, C), pw_ref[...],
                   preferred_element_type=f32) + pwb_ref[...]

    om = (jnp.dot(x_dw, Wom_ref[...], preferred_element_type=f32)
          + bom_ref[...]).reshape(BH, W, 2 * GK + GK)
    logits = om[..., 2 * GK:3 * GK]

    e_raw = jnp.exp(logits - jnp.max(logits, axis=-1, keepdims=True))
    j36 = jax.lax.broadcasted_iota(jnp.int32, (GK, GK), 0) // K2
    c36 = jax.lax.broadcasted_iota(jnp.int32, (GK, GK), 1) // K2
    sel36 = (j36 == c36).astype(f32)
    norm36 = jnp.dot(e_raw.reshape(BH * W, GK), sel36,
                     preferred_element_type=f32).reshape(BH, W, GK)
    e_attn = e_raw / norm36

    k_io = jax.lax.broadcasted_iota(jnp.int32, (1, 1, 2 * GK), 2)
    d_sel = k_io >= GK
    k9 = k_io % K2
    gcoord = jnp.where(d_sel, k9 % 3 - 1, k9 // 3 - 1).astype(f32)
    p = om[..., 0:2 * GK] * OFF_SCALE + gcoord

    whw = [jnp.maximum(1.0 - jnp.abs(p - d), 0.0)
           for d in (-2.0, -1.0, 0.0, 1.0, 2.0)]
    wh = [a[..., 0:GK] for a in whw]
    ww = [a[..., GK:2 * GK] for a in whw]

    j_io = jax.lax.broadcasted_iota(jnp.int32, (GK, C), 0) // K2
    c_io = jax.lax.broadcasted_iota(jnp.int32, (GK, C), 1) // GC
    sel = (j_io == c_io).astype(f32)
    dy_parts = []
    for dyi in range(5):
        awh = e_attn * wh[dyi]
        terms = []
        for dxi in range(5):
            t = (awh * ww[dxi]).reshape(BH * W, GK)
            wt96 = jnp.dot(t, sel, preferred_element_type=f32).reshape(BH, W, C)
            terms.append(wt96 * xpp[dyi:dyi + BH, dxi:dxi + W])
        dy_parts.append(((terms[0] + terms[1]) + (terms[2] + terms[3]))
                        + terms[4])
    out_pre = ((dy_parts[0] + dy_parts[1]) + (dy_parts[2] + dy_parts[3])) + dy_parts[4]
    out = (jnp.dot(out_pre.reshape(BH * W, C), Wout_ref[...],
                   preferred_element_type=f32) + bout_ref[...]).reshape(BH, W, C)
    o_ref[0] = out


def kernel(x, W_in, b_in, W_out, b_out, dw_w, dw_b, pw_w, pw_b,
           W_off, b_off, W_mask, b_mask):
    f32 = jnp.float32
    Wofp = W_off.reshape(C, G, K2, 2).transpose(0, 3, 1, 2).reshape(C, 2 * GK)
    bofp = b_off.reshape(G, K2, 2).transpose(2, 0, 1).reshape(2 * GK)
    Wom = jnp.concatenate([Wofp, W_mask], axis=1)
    bom = jnp.concatenate([bofp, b_mask]).reshape(1, 3 * GK)

    args = (
        x, x, x,
        W_in.astype(f32), b_in.reshape(1, C).astype(f32),
        dw_w.reshape(K2, C).astype(f32), dw_b.reshape(1, C).astype(f32),
        pw_w.reshape(C, C).astype(f32), pw_b.reshape(1, C).astype(f32),
        Wom.astype(f32), bom.astype(f32),
        W_out.astype(f32), b_out.reshape(1, C).astype(f32),
    )

    def full(a):
        r = a.ndim
        return pl.BlockSpec(a.shape, lambda b, i, _r=r: (0,) * _r)

    x_spec = lambda fn: pl.BlockSpec((1, BH, W, C), fn)
    in_specs = [
        x_spec(lambda b, i: (b, jnp.maximum(i - 1, 0), 0, 0)),
        x_spec(lambda b, i: (b, i, 0, 0)),
        x_spec(lambda b, i: (b, jnp.minimum(i + 1, N - 1), 0, 0)),
    ] + [full(a) for a in args[3:]]

    return pl.pallas_call(
        _dcn_kernel,
        grid=(B, N),
        in_specs=in_specs,
        out_specs=pl.BlockSpec((1, BH, W, C), lambda b, i: (b, i, 0, 0)),
        out_shape=jax.ShapeDtypeStruct((B, H, W, C), f32),
    )(*args)
